# Initial kernel scaffold; baseline (speedup 1.0000x reference)
import jax
import jax.numpy as jnp
from jax.experimental import pallas as pl


def kernel(x, table, Wih0f, Whh0f, bih0f, bhh0f, Wih0b, Whh0b, bih0b, bhh0b, Wih1f, Whh1f, bih1f, bhh1f, Wih1b, Whh1b, bih1b, bhh1b, fcW, fcb):
    raise NotImplementedError("write your pallas kernel here")


# speedup vs baseline: 2.8358x; 2.8358x over previous
import jax
import jax.numpy as jnp
from jax.experimental import pallas as pl
from jax.experimental.pallas import tpu as pltpu
from jax.experimental.pallas import tpu_sc as plsc

EMB = 128
HID = 512
B = 1024
T = 200
_GATHER_WIN = 128


def _gather(table_f32, idx_flat):
    n = idx_flat.shape[0]
    mesh = plsc.VectorSubcoreMesh(core_axis_name="core", subcore_axis_name="subcore")

    @pl.kernel(
        out_type=jax.ShapeDtypeStruct((n, EMB), table_f32.dtype),
        mesh=mesh,
    )
    def gather_kernel(tab_hbm, i_hbm, o_hbm):
        def body(i_vmem, o_vmem):
            pltpu.sync_copy(tab_hbm.at[i_vmem.at[0]], o_vmem)

        pltpu.emit_pipeline(
            body,
            grid=(n // _GATHER_WIN,),
            in_specs=[pl.BlockSpec((1, _GATHER_WIN), index_map=lambda i: (0, i))],
            out_specs=[pl.BlockSpec((_GATHER_WIN, EMB), index_map=lambda i: (i, 0))],
            core_axis_name=("core", "subcore"),
            dimension_semantics=(pltpu.PARALLEL,),
        )(i_hbm, o_hbm)

    return gather_kernel(table_f32, idx_flat.reshape(1, n))


def _lstm_step(g, c_ref, h_ref):
    i = jax.nn.sigmoid(g[:, :HID])
    f = jax.nn.sigmoid(g[:, HID : 2 * HID])
    gg = jnp.tanh(g[:, 2 * HID : 3 * HID])
    o = jax.nn.sigmoid(g[:, 3 * HID :])
    c = f * c_ref[...] + i * gg
    h = o * jnp.tanh(c)
    c_ref[...] = c
    h_ref[...] = h.astype(jnp.bfloat16)
    return h


def _dot(a, w_ref):
    return jnp.dot(a, w_ref[...], preferred_element_type=jnp.float32)


def _l0_body(sf, sb, wif, whf, bf, wib, whb, bb, of, ob,
             hf_s, cf_s, hb_s, cb_s):
    t = pl.program_id(0)

    @pl.when(t == 0)
    def _():
        hf_s[...] = jnp.zeros_like(hf_s)
        cf_s[...] = jnp.zeros_like(cf_s)
        hb_s[...] = jnp.zeros_like(hb_s)
        cb_s[...] = jnp.zeros_like(cb_s)

    gf = _dot(sf[0].astype(jnp.bfloat16), wif) + _dot(hf_s[...], whf) + bf[...]
    gb = _dot(sb[0].astype(jnp.bfloat16), wib) + _dot(hb_s[...], whb) + bb[...]
    hf = _lstm_step(gf, cf_s, hf_s)
    hb = _lstm_step(gb, cb_s, hb_s)
    of[0] = hf.astype(jnp.bfloat16)
    ob[0] = hb.astype(jnp.bfloat16)


def _layer0(seq, wif, whf, bf, wib, whb, bb):
    h_shape = jax.ShapeDtypeStruct((T, B, HID), jnp.bfloat16)
    wspec = pl.BlockSpec((EMB, 4 * HID), lambda t: (0, 0))
    hspec = pl.BlockSpec((HID, 4 * HID), lambda t: (0, 0))
    bspec = pl.BlockSpec((1, 4 * HID), lambda t: (0, 0))
    return pl.pallas_call(
        _l0_body,
        grid=(T,),
        in_specs=[
            pl.BlockSpec((1, B, EMB), lambda t: (t, 0, 0)),
            pl.BlockSpec((1, B, EMB), lambda t: (T - 1 - t, 0, 0)),
            wspec, hspec, bspec, wspec, hspec, bspec,
        ],
        out_specs=[
            pl.BlockSpec((1, B, HID), lambda t: (t, 0, 0)),
            pl.BlockSpec((1, B, HID), lambda t: (T - 1 - t, 0, 0)),
        ],
        out_shape=[h_shape, h_shape],
        scratch_shapes=[
            pltpu.VMEM((B, HID), jnp.bfloat16),
            pltpu.VMEM((B, HID), jnp.float32),
            pltpu.VMEM((B, HID), jnp.bfloat16),
            pltpu.VMEM((B, HID), jnp.float32),
        ],
        compiler_params=pltpu.CompilerParams(
            dimension_semantics=("arbitrary",),
        ),
    )(seq, seq, wif, whf, bf, wib, whb, bb)


def _l1_body(hf_f, hb_f, hf_b, hb_b,
             wifa, wifb, whf, bf, wiba, wibb, whb, bb, fcw, fcb,
             out, hf_s, cf_s, hb_s, cb_s):
    t = pl.program_id(0)

    @pl.when(t == 0)
    def _():
        hf_s[...] = jnp.zeros_like(hf_s)
        cf_s[...] = jnp.zeros_like(cf_s)
        hb_s[...] = jnp.zeros_like(hb_s)
        cb_s[...] = jnp.zeros_like(cb_s)

    gf = (_dot(hf_f[0], wifa) + _dot(hb_f[0], wifb)
          + _dot(hf_s[...], whf) + bf[...])
    gb = (_dot(hf_b[0], wiba) + _dot(hb_b[0], wibb)
          + _dot(hb_s[...], whb) + bb[...])
    hTf = _lstm_step(gf, cf_s, hf_s)
    hTb = _lstm_step(gb, cb_s, hb_s)

    @pl.when(t == T - 1)
    def _():
        logit = (
            jnp.sum(hTf * fcw[0:1, :HID], axis=1, keepdims=True)
            + jnp.sum(hTb * fcw[0:1, HID:], axis=1, keepdims=True)
            + fcb[...]
        )
        out[...] = jax.nn.sigmoid(logit)


def _layer1(hf, hb, wifa, wifb, whf, bf, wiba, wibb, whb, bb, fcw, fcb):
    inspec = pl.BlockSpec((1, B, HID), lambda t: (t, 0, 0))
    inspec_r = pl.BlockSpec((1, B, HID), lambda t: (T - 1 - t, 0, 0))
    wspec = pl.BlockSpec((HID, 4 * HID), lambda t: (0, 0))
    bspec = pl.BlockSpec((1, 4 * HID), lambda t: (0, 0))
    return pl.pallas_call(
        _l1_body,
        grid=(T,),
        in_specs=[
            inspec, inspec, inspec_r, inspec_r,
            wspec, wspec, wspec, bspec,
            wspec, wspec, wspec, bspec,
            pl.BlockSpec((1, 2 * HID), lambda t: (0, 0)),
            pl.BlockSpec((1, 1), lambda t: (0, 0)),
        ],
        out_specs=pl.BlockSpec((B, 1), lambda t: (0, 0)),
        out_shape=jax.ShapeDtypeStruct((B, 1), jnp.float32),
        scratch_shapes=[
            pltpu.VMEM((B, HID), jnp.bfloat16),
            pltpu.VMEM((B, HID), jnp.float32),
            pltpu.VMEM((B, HID), jnp.bfloat16),
            pltpu.VMEM((B, HID), jnp.float32),
        ],
        compiler_params=pltpu.CompilerParams(
            dimension_semantics=("arbitrary",),
        ),
    )(hf, hb, hf, hb, wifa, wifb, whf, bf, wiba, wibb, whb, bb, fcw, fcb)


def kernel(x, table, Wih0f, Whh0f, bih0f, bhh0f, Wih0b, Whh0b, bih0b, bhh0b,
           Wih1f, Whh1f, bih1f, bhh1f, Wih1b, Whh1b, bih1b, bhh1b, fcW, fcb):
    bf16 = jnp.bfloat16

    idx = jnp.transpose(x.astype(jnp.int32)).reshape(-1)
    emb = _gather(table, idx)
    seq = emb.reshape(T, B, EMB)

    w0f, u0f = Wih0f.T.astype(bf16), Whh0f.T.astype(bf16)
    w0b, u0b = Wih0b.T.astype(bf16), Whh0b.T.astype(bf16)
    b0f = (bih0f + bhh0f).reshape(1, 4 * HID)
    b0b = (bih0b + bhh0b).reshape(1, 4 * HID)

    hf, hb = _layer0(seq, w0f, u0f, b0f, w0b, u0b, b0b)

    w1f, u1f = Wih1f.T.astype(bf16), Whh1f.T.astype(bf16)
    w1b, u1b = Wih1b.T.astype(bf16), Whh1b.T.astype(bf16)
    b1f = (bih1f + bhh1f).reshape(1, 4 * HID)
    b1b = (bih1b + bhh1b).reshape(1, 4 * HID)

    return _layer1(
        hf, hb,
        w1f[:HID], w1f[HID:], u1f, b1f,
        w1b[:HID], w1b[HID:], u1b, b1b,
        fcW.reshape(1, 2 * HID), fcb.reshape(1, 1),
    )


# speedup vs baseline: 3.0607x; 1.0793x over previous
import jax
import jax.numpy as jnp
from jax.experimental import pallas as pl
from jax.experimental.pallas import tpu as pltpu
from jax.experimental.pallas import tpu_sc as plsc

EMB = 128
HID = 512
B = 1024
T = 200
_GATHER_WIN = 128


def _gather(table_f32, idx_flat):
    n = idx_flat.shape[0]
    mesh = plsc.VectorSubcoreMesh(core_axis_name="core", subcore_axis_name="subcore")

    @pl.kernel(
        out_type=jax.ShapeDtypeStruct((n, EMB), table_f32.dtype),
        mesh=mesh,
    )
    def gather_kernel(tab_hbm, i_hbm, o_hbm):
        def body(i_vmem, o_vmem):
            pltpu.sync_copy(tab_hbm.at[i_vmem.at[0]], o_vmem)

        pltpu.emit_pipeline(
            body,
            grid=(n // _GATHER_WIN,),
            in_specs=[pl.BlockSpec((1, _GATHER_WIN), index_map=lambda i: (0, i))],
            out_specs=[pl.BlockSpec((_GATHER_WIN, EMB), index_map=lambda i: (i, 0))],
            core_axis_name=("core", "subcore"),
            dimension_semantics=(pltpu.PARALLEL,),
        )(i_hbm, o_hbm)

    return gather_kernel(table_f32, idx_flat.reshape(1, n))


def _sig(x):
    return 0.5 + 0.5 * jnp.tanh(0.5 * x)


def _lstm_step(xc_ref, w_ref, b_ref, c_ref, h_lane0):
    g = jnp.dot(xc_ref[...], w_ref[...], preferred_element_type=jnp.float32)
    g += b_ref[...]
    i = _sig(g[:, :HID])
    f = _sig(g[:, HID : 2 * HID])
    gg = jnp.tanh(g[:, 2 * HID : 3 * HID])
    o = _sig(g[:, 3 * HID :])
    c = f * c_ref[...] + i * gg
    h = o * jnp.tanh(c)
    c_ref[...] = c
    xc_ref[:, h_lane0 : h_lane0 + HID] = h.astype(jnp.bfloat16)
    return h


def _l0_body(sf, sb, wf, bf, wb, bb, of, ob, xcf, cf, xcb, cb):
    t = pl.program_id(0)

    @pl.when(t == 0)
    def _():
        xcf[...] = jnp.zeros_like(xcf)
        cf[...] = jnp.zeros_like(cf)
        xcb[...] = jnp.zeros_like(xcb)
        cb[...] = jnp.zeros_like(cb)

    xcf[:, :EMB] = sf[0].astype(jnp.bfloat16)
    xcb[:, :EMB] = sb[0].astype(jnp.bfloat16)
    hf = _lstm_step(xcf, wf, bf, cf, EMB)
    hb = _lstm_step(xcb, wb, bb, cb, EMB)
    of[0] = hf.astype(jnp.bfloat16)
    ob[0] = hb.astype(jnp.bfloat16)


def _layer0(seq, wf, bf, wb, bb):
    h_shape = jax.ShapeDtypeStruct((T, B, HID), jnp.bfloat16)
    wspec = pl.BlockSpec((EMB + HID, 4 * HID), lambda t: (0, 0))
    bspec = pl.BlockSpec((1, 4 * HID), lambda t: (0, 0))
    return pl.pallas_call(
        _l0_body,
        grid=(T,),
        in_specs=[
            pl.BlockSpec((1, B, EMB), lambda t: (t, 0, 0)),
            pl.BlockSpec((1, B, EMB), lambda t: (T - 1 - t, 0, 0)),
            wspec, bspec, wspec, bspec,
        ],
        out_specs=[
            pl.BlockSpec((1, B, HID), lambda t: (t, 0, 0)),
            pl.BlockSpec((1, B, HID), lambda t: (T - 1 - t, 0, 0)),
        ],
        out_shape=[h_shape, h_shape],
        scratch_shapes=[
            pltpu.VMEM((B, EMB + HID), jnp.bfloat16),
            pltpu.VMEM((B, HID), jnp.float32),
            pltpu.VMEM((B, EMB + HID), jnp.bfloat16),
            pltpu.VMEM((B, HID), jnp.float32),
        ],
        compiler_params=pltpu.CompilerParams(
            dimension_semantics=("arbitrary",),
        ),
    )(seq, seq, wf, bf, wb, bb)


def _l1_body(hf_f, hb_f, hf_b, hb_b, wf, bf, wb, bb, fcw, fcb,
             out, xcf, cf, xcb, cb):
    t = pl.program_id(0)

    @pl.when(t == 0)
    def _():
        xcf[...] = jnp.zeros_like(xcf)
        cf[...] = jnp.zeros_like(cf)
        xcb[...] = jnp.zeros_like(xcb)
        cb[...] = jnp.zeros_like(cb)

    xcf[:, :HID] = hf_f[0]
    xcf[:, HID : 2 * HID] = hb_f[0]
    xcb[:, :HID] = hf_b[0]
    xcb[:, HID : 2 * HID] = hb_b[0]
    hTf = _lstm_step(xcf, wf, bf, cf, 2 * HID)
    hTb = _lstm_step(xcb, wb, bb, cb, 2 * HID)

    @pl.when(t == T - 1)
    def _():
        logit = (
            jnp.sum(hTf * fcw[0:1, :HID], axis=1, keepdims=True)
            + jnp.sum(hTb * fcw[0:1, HID:], axis=1, keepdims=True)
            + fcb[...]
        )
        out[...] = jax.nn.sigmoid(logit)


def _layer1(hf, hb, wf, bf, wb, bb, fcw, fcb):
    inspec = pl.BlockSpec((1, B, HID), lambda t: (t, 0, 0))
    inspec_r = pl.BlockSpec((1, B, HID), lambda t: (T - 1 - t, 0, 0))
    wspec = pl.BlockSpec((3 * HID, 4 * HID), lambda t: (0, 0))
    bspec = pl.BlockSpec((1, 4 * HID), lambda t: (0, 0))
    return pl.pallas_call(
        _l1_body,
        grid=(T,),
        in_specs=[
            inspec, inspec, inspec_r, inspec_r,
            wspec, bspec, wspec, bspec,
            pl.BlockSpec((1, 2 * HID), lambda t: (0, 0)),
            pl.BlockSpec((1, 1), lambda t: (0, 0)),
        ],
        out_specs=pl.BlockSpec((B, 1), lambda t: (0, 0)),
        out_shape=jax.ShapeDtypeStruct((B, 1), jnp.float32),
        scratch_shapes=[
            pltpu.VMEM((B, 3 * HID), jnp.bfloat16),
            pltpu.VMEM((B, HID), jnp.float32),
            pltpu.VMEM((B, 3 * HID), jnp.bfloat16),
            pltpu.VMEM((B, HID), jnp.float32),
        ],
        compiler_params=pltpu.CompilerParams(
            dimension_semantics=("arbitrary",),
        ),
    )(hf, hb, hf, hb, wf, bf, wb, bb, fcw, fcb)


def kernel(x, table, Wih0f, Whh0f, bih0f, bhh0f, Wih0b, Whh0b, bih0b, bhh0b,
           Wih1f, Whh1f, bih1f, bhh1f, Wih1b, Whh1b, bih1b, bhh1b, fcW, fcb):
    bf16 = jnp.bfloat16

    idx = jnp.transpose(x.astype(jnp.int32)).reshape(-1)
    emb = _gather(table, idx)
    seq = emb.reshape(T, B, EMB)

    w0f = jnp.concatenate([Wih0f.T, Whh0f.T], axis=0).astype(bf16)
    w0b = jnp.concatenate([Wih0b.T, Whh0b.T], axis=0).astype(bf16)
    b0f = (bih0f + bhh0f).reshape(1, 4 * HID)
    b0b = (bih0b + bhh0b).reshape(1, 4 * HID)

    hf, hb = _layer0(seq, w0f, b0f, w0b, b0b)

    w1f = jnp.concatenate([Wih1f.T, Whh1f.T], axis=0).astype(bf16)
    w1b = jnp.concatenate([Wih1b.T, Whh1b.T], axis=0).astype(bf16)
    b1f = (bih1f + bhh1f).reshape(1, 4 * HID)
    b1b = (bih1b + bhh1b).reshape(1, 4 * HID)

    return _layer1(
        hf, hb, w1f, b1f, w1b, b1b,
        fcW.reshape(1, 2 * HID), fcb.reshape(1, 1),
    )


# speedup vs baseline: 4.5159x; 1.4755x over previous
import jax
import jax.numpy as jnp
from jax.experimental import pallas as pl
from jax.experimental.pallas import tpu as pltpu
from jax.experimental.pallas import tpu_sc as plsc
from jax.sharding import Mesh, PartitionSpec as P

EMB = 128
HID = 512
T = 200
_GATHER_WIN = 128


def _gather(table_f32, idx_flat):
    n = idx_flat.shape[0]
    mesh = plsc.VectorSubcoreMesh(core_axis_name="core", subcore_axis_name="subcore")

    @pl.kernel(
        out_type=jax.ShapeDtypeStruct((n, EMB), table_f32.dtype),
        mesh=mesh,
    )
    def gather_kernel(tab_hbm, i_hbm, o_hbm):
        def body(i_vmem, o_vmem):
            pltpu.sync_copy(tab_hbm.at[i_vmem.at[0]], o_vmem)

        pltpu.emit_pipeline(
            body,
            grid=(n // _GATHER_WIN,),
            in_specs=[pl.BlockSpec((1, _GATHER_WIN), index_map=lambda i: (0, i))],
            out_specs=[pl.BlockSpec((_GATHER_WIN, EMB), index_map=lambda i: (i, 0))],
            core_axis_name=("core", "subcore"),
            dimension_semantics=(pltpu.PARALLEL,),
        )(i_hbm, o_hbm)

    return gather_kernel(table_f32, idx_flat.reshape(1, n))


def _sig(x):
    return 0.5 + 0.5 * jnp.tanh(0.5 * x)


def _pre(xc_ref, w_ref, b_ref):
    g = jnp.dot(xc_ref[...], w_ref[...], preferred_element_type=jnp.float32)
    return g + b_ref[...]


def _gates(g, c_ref, xc_ref, h_lane0):
    i = _sig(g[:, :HID])
    f = _sig(g[:, HID : 2 * HID])
    gg = jnp.tanh(g[:, 2 * HID : 3 * HID])
    o = _sig(g[:, 3 * HID :])
    c = f * c_ref[...] + i * gg
    h = o * jnp.tanh(c)
    c_ref[...] = c
    xc_ref[:, h_lane0 : h_lane0 + HID] = h.astype(jnp.bfloat16)
    return h


def _l0_body(sf, sb, wf, bf, wb, bb, of, ob, xcf, cf, xcb, cb):
    t = pl.program_id(0)

    @pl.when(t == 0)
    def _():
        xcf[...] = jnp.zeros_like(xcf)
        cf[...] = jnp.zeros_like(cf)
        xcb[...] = jnp.zeros_like(xcb)
        cb[...] = jnp.zeros_like(cb)

    xcf[:, :EMB] = sf[0].astype(jnp.bfloat16)
    xcb[:, :EMB] = sb[0].astype(jnp.bfloat16)
    gf = _pre(xcf, wf, bf)
    gb = _pre(xcb, wb, bb)
    hf = _gates(gf, cf, xcf, EMB)
    hb = _gates(gb, cb, xcb, EMB)
    of[0] = hf.astype(jnp.bfloat16)
    ob[0] = hb.astype(jnp.bfloat16)


def _layer0(seq, wf, bf, wb, bb):
    b_dim = seq.shape[1]
    h_shape = jax.ShapeDtypeStruct((T, b_dim, HID), jnp.bfloat16)
    wspec = pl.BlockSpec((EMB + HID, 4 * HID), lambda t: (0, 0))
    bspec = pl.BlockSpec((1, 4 * HID), lambda t: (0, 0))
    return pl.pallas_call(
        _l0_body,
        grid=(T,),
        in_specs=[
            pl.BlockSpec((1, b_dim, EMB), lambda t: (t, 0, 0)),
            pl.BlockSpec((1, b_dim, EMB), lambda t: (T - 1 - t, 0, 0)),
            wspec, bspec, wspec, bspec,
        ],
        out_specs=[
            pl.BlockSpec((1, b_dim, HID), lambda t: (t, 0, 0)),
            pl.BlockSpec((1, b_dim, HID), lambda t: (T - 1 - t, 0, 0)),
        ],
        out_shape=[h_shape, h_shape],
        scratch_shapes=[
            pltpu.VMEM((b_dim, EMB + HID), jnp.bfloat16),
            pltpu.VMEM((b_dim, HID), jnp.float32),
            pltpu.VMEM((b_dim, EMB + HID), jnp.bfloat16),
            pltpu.VMEM((b_dim, HID), jnp.float32),
        ],
        compiler_params=pltpu.CompilerParams(
            dimension_semantics=("arbitrary",),
        ),
    )(seq, seq, wf, bf, wb, bb)


def _l1_body(hf_f, hb_f, hf_b, hb_b, wf, bf, wb, bb, fcw, fcb,
             out, xcf, cf, xcb, cb):
    t = pl.program_id(0)

    @pl.when(t == 0)
    def _():
        xcf[...] = jnp.zeros_like(xcf)
        cf[...] = jnp.zeros_like(cf)
        xcb[...] = jnp.zeros_like(xcb)
        cb[...] = jnp.zeros_like(cb)

    xcf[:, :HID] = hf_f[0]
    xcf[:, HID : 2 * HID] = hb_f[0]
    xcb[:, :HID] = hf_b[0]
    xcb[:, HID : 2 * HID] = hb_b[0]
    gf = _pre(xcf, wf, bf)
    gb = _pre(xcb, wb, bb)
    hTf = _gates(gf, cf, xcf, 2 * HID)
    hTb = _gates(gb, cb, xcb, 2 * HID)

    @pl.when(t == T - 1)
    def _():
        logit = (
            jnp.sum(hTf * fcw[0:1, :HID], axis=1, keepdims=True)
            + jnp.sum(hTb * fcw[0:1, HID:], axis=1, keepdims=True)
            + fcb[...]
        )
        out[...] = jax.nn.sigmoid(logit)


def _layer1(hf, hb, wf, bf, wb, bb, fcw, fcb):
    b_dim = hf.shape[1]
    inspec = pl.BlockSpec((1, b_dim, HID), lambda t: (t, 0, 0))
    inspec_r = pl.BlockSpec((1, b_dim, HID), lambda t: (T - 1 - t, 0, 0))
    wspec = pl.BlockSpec((3 * HID, 4 * HID), lambda t: (0, 0))
    bspec = pl.BlockSpec((1, 4 * HID), lambda t: (0, 0))
    return pl.pallas_call(
        _l1_body,
        grid=(T,),
        in_specs=[
            inspec, inspec, inspec_r, inspec_r,
            wspec, bspec, wspec, bspec,
            pl.BlockSpec((1, 2 * HID), lambda t: (0, 0)),
            pl.BlockSpec((1, 1), lambda t: (0, 0)),
        ],
        out_specs=pl.BlockSpec((b_dim, 1), lambda t: (0, 0)),
        out_shape=jax.ShapeDtypeStruct((b_dim, 1), jnp.float32),
        scratch_shapes=[
            pltpu.VMEM((b_dim, 3 * HID), jnp.bfloat16),
            pltpu.VMEM((b_dim, HID), jnp.float32),
            pltpu.VMEM((b_dim, 3 * HID), jnp.bfloat16),
            pltpu.VMEM((b_dim, HID), jnp.float32),
        ],
        compiler_params=pltpu.CompilerParams(
            dimension_semantics=("arbitrary",),
        ),
    )(hf, hb, hf, hb, wf, bf, wb, bb, fcw, fcb)


def _run(x, table, Wih0f, Whh0f, bih0f, bhh0f, Wih0b, Whh0b, bih0b, bhh0b,
         Wih1f, Whh1f, bih1f, bhh1f, Wih1b, Whh1b, bih1b, bhh1b, fcW, fcb):
    bf16 = jnp.bfloat16
    b_dim = x.shape[0]

    idx = jnp.transpose(x.astype(jnp.int32)).reshape(-1)
    emb = _gather(table, idx)
    seq = emb.reshape(T, b_dim, EMB)

    w0f = jnp.concatenate([Wih0f.T, Whh0f.T], axis=0).astype(bf16)
    w0b = jnp.concatenate([Wih0b.T, Whh0b.T], axis=0).astype(bf16)
    b0f = (bih0f + bhh0f).reshape(1, 4 * HID)
    b0b = (bih0b + bhh0b).reshape(1, 4 * HID)

    hf, hb = _layer0(seq, w0f, b0f, w0b, b0b)

    w1f = jnp.concatenate([Wih1f.T, Whh1f.T], axis=0).astype(bf16)
    w1b = jnp.concatenate([Wih1b.T, Whh1b.T], axis=0).astype(bf16)
    b1f = (bih1f + bhh1f).reshape(1, 4 * HID)
    b1b = (bih1b + bhh1b).reshape(1, 4 * HID)

    return _layer1(
        hf, hb, w1f, b1f, w1b, b1b,
        fcW.reshape(1, 2 * HID), fcb.reshape(1, 1),
    )


def kernel(x, table, Wih0f, Whh0f, bih0f, bhh0f, Wih0b, Whh0b, bih0b, bhh0b,
           Wih1f, Whh1f, bih1f, bhh1f, Wih1b, Whh1b, bih1b, bhh1b, fcW, fcb):
    args = (x, table, Wih0f, Whh0f, bih0f, bhh0f, Wih0b, Whh0b, bih0b, bhh0b,
            Wih1f, Whh1f, bih1f, bhh1f, Wih1b, Whh1b, bih1b, bhh1b, fcW, fcb)
    devs = jax.devices()
    n_shards = 2 if (len(devs) >= 2 and x.shape[0] % 2 == 0) else 1
    if n_shards == 1:
        return _run(*args)
    mesh = Mesh(devs[:n_shards], ("d",))
    in_specs = (P("d", None),) + (P(),) * 19
    fn = jax.shard_map(_run, mesh=mesh, in_specs=in_specs,
                       out_specs=P("d", None), check_vma=False)
    return fn(*args)


# speedup vs baseline: 4.8282x; 1.0691x over previous
import jax
import jax.numpy as jnp
from jax.experimental import pallas as pl
from jax.experimental.pallas import tpu as pltpu
from jax.experimental.pallas import tpu_sc as plsc
from jax.sharding import Mesh, PartitionSpec as P

EMB = 128
HID = 512
T = 200
_GATHER_WIN = 128


def _gather(table_f32, idx_flat, num_cores):
    n = idx_flat.shape[0]
    mesh = plsc.VectorSubcoreMesh(core_axis_name="core", subcore_axis_name="subcore",
                                  num_cores=num_cores)

    @pl.kernel(
        out_type=jax.ShapeDtypeStruct((n, EMB), table_f32.dtype),
        mesh=mesh,
    )
    def gather_kernel(tab_hbm, i_hbm, o_hbm):
        def body(i_vmem, o_vmem):
            pltpu.sync_copy(tab_hbm.at[i_vmem.at[0]], o_vmem)

        pltpu.emit_pipeline(
            body,
            grid=(n // _GATHER_WIN,),
            in_specs=[pl.BlockSpec((1, _GATHER_WIN), index_map=lambda i: (0, i))],
            out_specs=[pl.BlockSpec((_GATHER_WIN, EMB), index_map=lambda i: (i, 0))],
            core_axis_name=("core", "subcore") if num_cores > 1 else "subcore",
            dimension_semantics=(pltpu.PARALLEL,),
        )(i_hbm, o_hbm)

    return gather_kernel(table_f32, idx_flat.reshape(1, n))


def _sig(x):
    return 0.5 + 0.5 * jnp.tanh(0.5 * x)


def _pre(xc_ref, w_ref, b_ref):
    g = jnp.dot(xc_ref[...], w_ref[...], preferred_element_type=jnp.float32)
    return g + b_ref[...]


def _gates(g, c_ref, xc_ref, h_lane0):
    i = _sig(g[:, :HID])
    f = _sig(g[:, HID : 2 * HID])
    gg = jnp.tanh(g[:, 2 * HID : 3 * HID])
    o = _sig(g[:, 3 * HID :])
    c = f * c_ref[...] + i * gg
    h = o * jnp.tanh(c)
    c_ref[...] = c
    xc_ref[:, h_lane0 : h_lane0 + HID] = h.astype(jnp.bfloat16)
    return h


def _l0_body(sf, sb, wf, bf, wb, bb, of, ob, xcf, cf, xcb, cb):
    t = pl.program_id(0)

    @pl.when(t == 0)
    def _():
        xcf[...] = jnp.zeros_like(xcf)
        cf[...] = jnp.zeros_like(cf)
        xcb[...] = jnp.zeros_like(xcb)
        cb[...] = jnp.zeros_like(cb)

    xcf[:, :EMB] = sf[0].astype(jnp.bfloat16)
    xcb[:, :EMB] = sb[0].astype(jnp.bfloat16)
    gf = _pre(xcf, wf, bf)
    gb = _pre(xcb, wb, bb)
    hf = _gates(gf, cf, xcf, EMB)
    hb = _gates(gb, cb, xcb, EMB)
    of[0] = hf.astype(jnp.bfloat16)
    ob[0] = hb.astype(jnp.bfloat16)


def _layer0(seq, wf, bf, wb, bb):
    b_dim = seq.shape[1]
    h_shape = jax.ShapeDtypeStruct((T, b_dim, HID), jnp.bfloat16)
    wspec = pl.BlockSpec((EMB + HID, 4 * HID), lambda t: (0, 0))
    bspec = pl.BlockSpec((1, 4 * HID), lambda t: (0, 0))
    return pl.pallas_call(
        _l0_body,
        grid=(T,),
        in_specs=[
            pl.BlockSpec((1, b_dim, EMB), lambda t: (t, 0, 0)),
            pl.BlockSpec((1, b_dim, EMB), lambda t: (T - 1 - t, 0, 0)),
            wspec, bspec, wspec, bspec,
        ],
        out_specs=[
            pl.BlockSpec((1, b_dim, HID), lambda t: (t, 0, 0)),
            pl.BlockSpec((1, b_dim, HID), lambda t: (T - 1 - t, 0, 0)),
        ],
        out_shape=[h_shape, h_shape],
        scratch_shapes=[
            pltpu.VMEM((b_dim, EMB + HID), jnp.bfloat16),
            pltpu.VMEM((b_dim, HID), jnp.float32),
            pltpu.VMEM((b_dim, EMB + HID), jnp.bfloat16),
            pltpu.VMEM((b_dim, HID), jnp.float32),
        ],
        compiler_params=pltpu.CompilerParams(
            dimension_semantics=("arbitrary",),
        ),
    )(seq, seq, wf, bf, wb, bb)


def _l1_body(hf_f, hb_f, hf_b, hb_b, wf, bf, wb, bb, fcw, fcb,
             out, xcf, cf, xcb, cb):
    t = pl.program_id(0)

    @pl.when(t == 0)
    def _():
        xcf[...] = jnp.zeros_like(xcf)
        cf[...] = jnp.zeros_like(cf)
        xcb[...] = jnp.zeros_like(xcb)
        cb[...] = jnp.zeros_like(cb)

    xcf[:, :HID] = hf_f[0]
    xcf[:, HID : 2 * HID] = hb_f[0]
    xcb[:, :HID] = hf_b[0]
    xcb[:, HID : 2 * HID] = hb_b[0]
    gf = _pre(xcf, wf, bf)
    gb = _pre(xcb, wb, bb)
    hTf = _gates(gf, cf, xcf, 2 * HID)
    hTb = _gates(gb, cb, xcb, 2 * HID)

    @pl.when(t == T - 1)
    def _():
        logit = (
            jnp.sum(hTf * fcw[0:1, :HID], axis=1, keepdims=True)
            + jnp.sum(hTb * fcw[0:1, HID:], axis=1, keepdims=True)
            + fcb[...]
        )
        out[...] = jax.nn.sigmoid(logit)


def _layer1(hf, hb, wf, bf, wb, bb, fcw, fcb):
    b_dim = hf.shape[1]
    inspec = pl.BlockSpec((1, b_dim, HID), lambda t: (t, 0, 0))
    inspec_r = pl.BlockSpec((1, b_dim, HID), lambda t: (T - 1 - t, 0, 0))
    wspec = pl.BlockSpec((3 * HID, 4 * HID), lambda t: (0, 0))
    bspec = pl.BlockSpec((1, 4 * HID), lambda t: (0, 0))
    return pl.pallas_call(
        _l1_body,
        grid=(T,),
        in_specs=[
            inspec, inspec, inspec_r, inspec_r,
            wspec, bspec, wspec, bspec,
            pl.BlockSpec((1, 2 * HID), lambda t: (0, 0)),
            pl.BlockSpec((1, 1), lambda t: (0, 0)),
        ],
        out_specs=pl.BlockSpec((b_dim, 1), lambda t: (0, 0)),
        out_shape=jax.ShapeDtypeStruct((b_dim, 1), jnp.float32),
        scratch_shapes=[
            pltpu.VMEM((b_dim, 3 * HID), jnp.bfloat16),
            pltpu.VMEM((b_dim, HID), jnp.float32),
            pltpu.VMEM((b_dim, 3 * HID), jnp.bfloat16),
            pltpu.VMEM((b_dim, HID), jnp.float32),
        ],
        compiler_params=pltpu.CompilerParams(
            dimension_semantics=("arbitrary",),
        ),
    )(hf, hb, hf, hb, wf, bf, wb, bb, fcw, fcb)


def _run(x, table, Wih0f, Whh0f, bih0f, bhh0f, Wih0b, Whh0b, bih0b, bhh0b,
         Wih1f, Whh1f, bih1f, bhh1f, Wih1b, Whh1b, bih1b, bhh1b, fcW, fcb,
         sc_cores=2):
    bf16 = jnp.bfloat16
    b_dim = x.shape[0]

    idx = jnp.transpose(x.astype(jnp.int32)).reshape(-1)
    emb = _gather(table, idx, sc_cores)
    seq = emb.reshape(T, b_dim, EMB)

    w0f = jnp.concatenate([Wih0f.T, Whh0f.T], axis=0).astype(bf16)
    w0b = jnp.concatenate([Wih0b.T, Whh0b.T], axis=0).astype(bf16)
    b0f = (bih0f + bhh0f).reshape(1, 4 * HID)
    b0b = (bih0b + bhh0b).reshape(1, 4 * HID)

    hf, hb = _layer0(seq, w0f, b0f, w0b, b0b)

    w1f = jnp.concatenate([Wih1f.T, Whh1f.T], axis=0).astype(bf16)
    w1b = jnp.concatenate([Wih1b.T, Whh1b.T], axis=0).astype(bf16)
    b1f = (bih1f + bhh1f).reshape(1, 4 * HID)
    b1b = (bih1b + bhh1b).reshape(1, 4 * HID)

    return _layer1(
        hf, hb, w1f, b1f, w1b, b1b,
        fcW.reshape(1, 2 * HID), fcb.reshape(1, 1),
    )


def kernel(x, table, Wih0f, Whh0f, bih0f, bhh0f, Wih0b, Whh0b, bih0b, bhh0b,
           Wih1f, Whh1f, bih1f, bhh1f, Wih1b, Whh1b, bih1b, bhh1b, fcW, fcb):
    args = (x, table, Wih0f, Whh0f, bih0f, bhh0f, Wih0b, Whh0b, bih0b, bhh0b,
            Wih1f, Whh1f, bih1f, bhh1f, Wih1b, Whh1b, bih1b, bhh1b, fcW, fcb)
    devs = jax.devices()
    n_shards = 2 if (len(devs) >= 2 and x.shape[0] % 2 == 0) else 1
    if n_shards == 1:
        return _run(*args)
    mesh = Mesh(devs[:n_shards], ("d",))
    in_specs = (P("d", None),) + (P(),) * 19

    def _run_shard(*a):
        return _run(*a, sc_cores=1)

    fn = jax.shard_map(_run_shard, mesh=mesh, in_specs=in_specs,
                       out_specs=P("d", None), check_vma=False)
    return fn(*args)
